# baseline (speedup 1.0000x reference)
import functools

import jax
import jax.numpy as jnp
from jax import lax
from jax.experimental import pallas as pl
from jax.experimental.pallas import tpu as pltpu
from jax.experimental.pallas import tpu_sc as plsc

BATCH = 16384
FIELDS = 26
TOTAL = BATCH * FIELDS

NUM_CORES = 2
NUM_SUBCORES = 16
NUM_WORKERS = NUM_CORES * NUM_SUBCORES
B_PER_W = TOTAL // NUM_WORKERS

_mesh = plsc.VectorSubcoreMesh(core_axis_name="c", subcore_axis_name="s")


@functools.partial(
    pl.kernel,
    mesh=_mesh,
    out_type=jax.ShapeDtypeStruct((TOTAL,), jnp.float32),
    scratch_types=[
        pltpu.VMEM((B_PER_W,), jnp.int32),
        pltpu.VMEM((B_PER_W,), jnp.float32),
        pltpu.SemaphoreType.DMA,
    ],
)
def _gather_kernel(w_hbm, x_hbm, out_hbm, idx_v, vals_v, sem):
    wid = lax.axis_index("s") * NUM_CORES + lax.axis_index("c")
    base = wid * B_PER_W
    pltpu.sync_copy(x_hbm.at[pl.ds(base, B_PER_W)], idx_v)
    pltpu.async_copy(w_hbm.at[idx_v], vals_v, sem).wait()
    pltpu.sync_copy(vals_v, out_hbm.at[pl.ds(base, B_PER_W)])


def kernel(x, weights):
    flat = x.reshape(TOTAL).astype(jnp.int32)
    out = _gather_kernel(weights, flat)
    return out.reshape(BATCH, FIELDS)


# speedup vs baseline: 1.1930x; 1.1930x over previous
import functools

import jax
import jax.numpy as jnp
from jax import lax
from jax.experimental import pallas as pl
from jax.experimental.pallas import tpu as pltpu
from jax.experimental.pallas import tpu_sc as plsc

BATCH = 16384
FIELDS = 26
TOTAL = BATCH * FIELDS

NUM_CORES = 2
NUM_SUBCORES = 16
NUM_WORKERS = NUM_CORES * NUM_SUBCORES
ROWS_PER_W = BATCH // NUM_WORKERS
B_PER_W = ROWS_PER_W * FIELDS

LANES = 16
PAT = 13
GROUP_ROWS = 8
GROUP_FLAT = GROUP_ROWS * FIELDS
GROUPS = ROWS_PER_W // GROUP_ROWS
OUT_ROWS = 256
OUT_GROUPS = OUT_ROWS // GROUP_ROWS
OUT_FLAT = OUT_ROWS * FIELDS

_mesh = plsc.VectorSubcoreMesh(core_axis_name="c", subcore_axis_name="s")


@functools.partial(
    pl.kernel,
    mesh=_mesh,
    out_type=jax.ShapeDtypeStruct((BATCH, FIELDS), jnp.float32),
    scratch_types=[
        pltpu.VMEM((ROWS_PER_W, FIELDS), jnp.int32),
        pltpu.VMEM((B_PER_W,), jnp.int32),
        pltpu.VMEM((B_PER_W,), jnp.float32),
        pltpu.VMEM((OUT_ROWS, FIELDS), jnp.float32),
        pltpu.SemaphoreType.DMA,
    ],
    compiler_params=pltpu.CompilerParams(
        use_tc_tiling_on_sc=True, needs_layout_passes=False
    ),
)
def _gather_kernel(w_hbm, x_hbm, out_hbm, xin_v, idx_v, vals_v, xout_v, sem):
    wid = lax.axis_index("s") * NUM_CORES + lax.axis_index("c")
    row0 = wid * ROWS_PER_W

    pltpu.sync_copy(x_hbm.at[pl.ds(row0, ROWS_PER_W), :], xin_v)

    lane = lax.iota(jnp.int32, LANES)
    rs, cs = [], []
    for j in range(PAT):
        p = lane + j * LANES
        rs.append(lax.div(p, FIELDS))
        cs.append(lax.rem(p, FIELDS))

    def compact_group(t, carry):
        base_r = t * GROUP_ROWS
        base_f = t * GROUP_FLAT
        for j in range(PAT):
            v = plsc.load_gather(xin_v, [base_r + rs[j], cs[j]])
            idx_v[pl.ds(base_f + j * LANES, LANES)] = v
        return carry

    lax.fori_loop(0, GROUPS, compact_group, 0)

    pltpu.async_copy(w_hbm.at[idx_v], vals_v, sem).wait()

    def expand_group(half):
        def body(t, carry):
            base_r = t * GROUP_ROWS
            base_f = half * OUT_FLAT + t * GROUP_FLAT
            for j in range(PAT):
                v = vals_v[pl.ds(base_f + j * LANES, LANES)]
                plsc.store_scatter(xout_v, [base_r + rs[j], cs[j]], v)
            return carry
        return body

    for half in range(2):
        lax.fori_loop(0, OUT_GROUPS, expand_group(half), 0)
        pltpu.sync_copy(
            xout_v, out_hbm.at[pl.ds(row0 + half * OUT_ROWS, OUT_ROWS), :]
        )


def kernel(x, weights):
    return _gather_kernel(weights, x.astype(jnp.int32))


# speedup vs baseline: 1.2870x; 1.0788x over previous
import functools

import jax
import jax.numpy as jnp
from jax import lax
from jax.experimental import pallas as pl
from jax.experimental.pallas import tpu as pltpu
from jax.experimental.pallas import tpu_sc as plsc

BATCH = 16384
FIELDS = 26
TOTAL = BATCH * FIELDS

NUM_CORES = 2
NUM_SUBCORES = 16
NUM_WORKERS = NUM_CORES * NUM_SUBCORES
COLS_PER_W = BATCH // NUM_WORKERS
B_PER_W = COLS_PER_W * FIELDS

LANES = 16
PAT = 13
GROUP_COLS = 8
GROUP_FLAT = GROUP_COLS * FIELDS
GROUPS = COLS_PER_W // GROUP_COLS

_mesh = plsc.VectorSubcoreMesh(core_axis_name="c", subcore_axis_name="s")


@functools.partial(
    pl.kernel,
    mesh=_mesh,
    out_type=jax.ShapeDtypeStruct((FIELDS, BATCH), jnp.float32),
    scratch_types=[
        pltpu.VMEM((FIELDS, COLS_PER_W), jnp.int32),
        pltpu.VMEM((B_PER_W,), jnp.int32),
        pltpu.VMEM((B_PER_W,), jnp.float32),
        pltpu.VMEM((FIELDS, COLS_PER_W), jnp.float32),
        pltpu.SemaphoreType.DMA,
    ],
    compiler_params=pltpu.CompilerParams(
        use_tc_tiling_on_sc=True, needs_layout_passes=False
    ),
)
def _gather_kernel(w_hbm, xt_hbm, out_hbm, xin_v, idx_v, vals_v, xout_v, sem):
    wid = lax.axis_index("s") * NUM_CORES + lax.axis_index("c")
    col0 = wid * COLS_PER_W

    pltpu.sync_copy(xt_hbm.at[:, pl.ds(col0, COLS_PER_W)], xin_v)

    lane = lax.iota(jnp.int32, LANES)
    fs, cs = [], []
    for j in range(PAT):
        p = lane + j * LANES
        cs.append(lax.div(p, FIELDS))
        fs.append(lax.rem(p, FIELDS))

    def compact_group(t, carry):
        base_c = t * GROUP_COLS
        base_f = t * GROUP_FLAT
        for j in range(PAT):
            v = plsc.load_gather(xin_v, [fs[j], base_c + cs[j]])
            idx_v[pl.ds(base_f + j * LANES, LANES)] = v
        return carry

    lax.fori_loop(0, GROUPS, compact_group, 0)

    pltpu.async_copy(w_hbm.at[idx_v], vals_v, sem).wait()

    def expand_group(t, carry):
        base_c = t * GROUP_COLS
        base_f = t * GROUP_FLAT
        for j in range(PAT):
            v = vals_v[pl.ds(base_f + j * LANES, LANES)]
            plsc.store_scatter(xout_v, [fs[j], base_c + cs[j]], v)
        return carry

    lax.fori_loop(0, GROUPS, expand_group, 0)

    pltpu.sync_copy(xout_v, out_hbm.at[:, pl.ds(col0, COLS_PER_W)])


def kernel(x, weights):
    out_t = _gather_kernel(weights, x.astype(jnp.int32).T)
    return out_t.T


# speedup vs baseline: 1.5861x; 1.2324x over previous
import functools

import jax
import jax.numpy as jnp
from jax import lax
from jax.experimental import pallas as pl
from jax.experimental.pallas import tpu as pltpu
from jax.experimental.pallas import tpu_sc as plsc

BATCH = 16384
FIELDS = 26

NUM_CORES = 2
NUM_SUBCORES = 16
NUM_WORKERS = NUM_CORES * NUM_SUBCORES
COLS_PER_W = BATCH // NUM_WORKERS
B_PER_W = COLS_PER_W * FIELDS

LANES = 16
VECS_PER_ROW = COLS_PER_W // LANES

_mesh = plsc.VectorSubcoreMesh(core_axis_name="c", subcore_axis_name="s")


@functools.partial(
    pl.kernel,
    mesh=_mesh,
    out_type=jax.ShapeDtypeStruct((FIELDS, BATCH), jnp.float32),
    scratch_types=[
        pltpu.VMEM((FIELDS, COLS_PER_W), jnp.int32),
        pltpu.VMEM((B_PER_W,), jnp.int32),
        pltpu.VMEM((B_PER_W,), jnp.float32),
        pltpu.VMEM((FIELDS, COLS_PER_W), jnp.float32),
        pltpu.SemaphoreType.DMA,
    ],
    compiler_params=pltpu.CompilerParams(
        use_tc_tiling_on_sc=True, needs_layout_passes=False
    ),
)
def _gather_kernel(w_hbm, xt_hbm, out_hbm, xin_v, idx_v, vals_v, xout_v, sem):
    wid = lax.axis_index("s") * NUM_CORES + lax.axis_index("c")
    col0 = wid * COLS_PER_W

    pltpu.sync_copy(xt_hbm.at[:, pl.ds(col0, COLS_PER_W)], xin_v)

    def compact_row(f, carry):
        base = f * COLS_PER_W
        for v in range(VECS_PER_ROW):
            idx_v[pl.ds(base + v * LANES, LANES)] = xin_v[f, pl.ds(v * LANES, LANES)]
        return carry

    lax.fori_loop(0, FIELDS, compact_row, 0)

    pltpu.async_copy(w_hbm.at[idx_v], vals_v, sem).wait()

    def expand_row(f, carry):
        base = f * COLS_PER_W
        for v in range(VECS_PER_ROW):
            xout_v[f, pl.ds(v * LANES, LANES)] = vals_v[pl.ds(base + v * LANES, LANES)]
        return carry

    lax.fori_loop(0, FIELDS, expand_row, 0)

    pltpu.sync_copy(xout_v, out_hbm.at[:, pl.ds(col0, COLS_PER_W)])


def kernel(x, weights):
    out_t = _gather_kernel(weights, x.astype(jnp.int32).T)
    return out_t.T


# speedup vs baseline: 1.6988x; 1.0711x over previous
import functools

import jax
import jax.numpy as jnp
from jax import lax
from jax.experimental import pallas as pl
from jax.experimental.pallas import tpu as pltpu
from jax.experimental.pallas import tpu_sc as plsc

BATCH = 16384
FIELDS = 26

NUM_CORES = 2
NUM_SUBCORES = 16
NUM_WORKERS = NUM_CORES * NUM_SUBCORES
COLS_PER_W = BATCH // NUM_WORKERS
B_PER_W = COLS_PER_W * FIELDS

LANES = 16
NCHUNK = 4
CHUNK_COLS = COLS_PER_W // NCHUNK
CHUNK_FLAT = CHUNK_COLS * FIELDS
VECS_PER_ROW = CHUNK_COLS // LANES

_mesh = plsc.VectorSubcoreMesh(core_axis_name="c", subcore_axis_name="s")


@functools.partial(
    pl.kernel,
    mesh=_mesh,
    out_type=jax.ShapeDtypeStruct((FIELDS, BATCH), jnp.float32),
    scratch_types=[
        pltpu.VMEM((FIELDS, COLS_PER_W), jnp.int32),
        pltpu.VMEM((B_PER_W,), jnp.int32),
        pltpu.VMEM((B_PER_W,), jnp.float32),
        pltpu.VMEM((FIELDS, COLS_PER_W), jnp.float32),
        pltpu.SemaphoreType.DMA,
        pltpu.SemaphoreType.DMA,
        pltpu.SemaphoreType.DMA,
    ],
    compiler_params=pltpu.CompilerParams(
        use_tc_tiling_on_sc=True, needs_layout_passes=False
    ),
)
def _gather_kernel(
    w_hbm, xt_hbm, out_hbm, xin_v, idx_v, vals_v, xout_v, sem_in, sem_g, sem_out
):
    wid = lax.axis_index("s") * NUM_CORES + lax.axis_index("c")
    col0 = wid * COLS_PER_W

    in_copies = [
        pltpu.async_copy(
            xt_hbm.at[:, pl.ds(col0 + q * CHUNK_COLS, CHUNK_COLS)],
            xin_v.at[:, pl.ds(q * CHUNK_COLS, CHUNK_COLS)],
            sem_in,
        )
        for q in range(NCHUNK)
    ]

    gathers = []
    for q in range(NCHUNK):
        in_copies[q].wait()
        cbase = q * CHUNK_COLS
        fbase = q * CHUNK_FLAT

        def compact_row(f, carry, cbase=cbase, fbase=fbase):
            base = fbase + f * CHUNK_COLS
            for v in range(VECS_PER_ROW):
                idx_v[pl.ds(base + v * LANES, LANES)] = xin_v[
                    f, pl.ds(cbase + v * LANES, LANES)
                ]
            return carry

        lax.fori_loop(0, FIELDS, compact_row, 0)
        gathers.append(
            pltpu.async_copy(
                w_hbm.at[idx_v.at[pl.ds(fbase, CHUNK_FLAT)]],
                vals_v.at[pl.ds(fbase, CHUNK_FLAT)],
                sem_g,
            )
        )

    out_copies = []
    for q in range(NCHUNK):
        gathers[q].wait()
        cbase = q * CHUNK_COLS
        fbase = q * CHUNK_FLAT

        def expand_row(f, carry, cbase=cbase, fbase=fbase):
            base = fbase + f * CHUNK_COLS
            for v in range(VECS_PER_ROW):
                xout_v[f, pl.ds(cbase + v * LANES, LANES)] = vals_v[
                    pl.ds(base + v * LANES, LANES)
                ]
            return carry

        lax.fori_loop(0, FIELDS, expand_row, 0)
        out_copies.append(
            pltpu.async_copy(
                xout_v.at[:, pl.ds(cbase, CHUNK_COLS)],
                out_hbm.at[:, pl.ds(col0 + cbase, CHUNK_COLS)],
                sem_out,
            )
        )

    for c in out_copies:
        c.wait()


def kernel(x, weights):
    out_t = _gather_kernel(weights, x.astype(jnp.int32).T)
    return out_t.T


# speedup vs baseline: 2.1056x; 1.2394x over previous
import functools

import jax
import jax.numpy as jnp
from jax import lax
from jax.experimental import pallas as pl
from jax.experimental.pallas import tpu as pltpu
from jax.experimental.pallas import tpu_sc as plsc

BATCH = 16384
FIELDS = 26
VOCAB = 1000001

NUM_CORES = 2
NUM_SUBCORES = 16
NUM_WORKERS = NUM_CORES * NUM_SUBCORES
COLS_PER_W = BATCH // NUM_WORKERS
B_PER_W = COLS_PER_W * FIELDS

LANES = 16
NCHUNK = 4
CHUNK_COLS = COLS_PER_W // NCHUNK
CHUNK_FLAT = CHUNK_COLS * FIELDS
VECS_PER_ROW = CHUNK_COLS // LANES

STAGE_CHUNK = 62592
STAGE_TAIL = 61056
STAGE_REM_OFF = 15 * STAGE_CHUNK + STAGE_TAIL
STAGE_REM = VOCAB - STAGE_REM_OFF

_mesh = plsc.VectorSubcoreMesh(core_axis_name="c", subcore_axis_name="s")


@functools.partial(
    pl.kernel,
    mesh=_mesh,
    out_type=jax.ShapeDtypeStruct((FIELDS, BATCH), jnp.float32),
    scratch_types=[
        pltpu.VMEM_SHARED((VOCAB,), jnp.float32),
        pltpu.VMEM((FIELDS, COLS_PER_W), jnp.int32),
        pltpu.VMEM((B_PER_W,), jnp.int32),
        pltpu.VMEM((B_PER_W,), jnp.float32),
        pltpu.VMEM((FIELDS, COLS_PER_W), jnp.float32),
        pltpu.VMEM((128,), jnp.float32),
        pltpu.SemaphoreType.DMA,
        pltpu.SemaphoreType.DMA,
        pltpu.SemaphoreType.DMA,
    ],
    compiler_params=pltpu.CompilerParams(
        use_tc_tiling_on_sc=True, needs_layout_passes=False
    ),
)
def _gather_kernel(
    w_hbm, xt_hbm, out_hbm,
    table_s, xin_v, idx_v, vals_v, xout_v, tail_v,
    sem_in, sem_g, sem_out,
):
    cid = lax.axis_index("c")
    sid = lax.axis_index("s")
    wid = sid * NUM_CORES + cid
    col0 = wid * COLS_PER_W

    in_copies = [
        pltpu.async_copy(
            xt_hbm.at[:, pl.ds(col0 + q * CHUNK_COLS, CHUNK_COLS)],
            xin_v.at[:, pl.ds(q * CHUNK_COLS, CHUNK_COLS)],
            sem_in,
        )
        for q in range(NCHUNK)
    ]

    @pl.when(sid < NUM_SUBCORES - 1)
    def _():
        off = pl.multiple_of(sid * STAGE_CHUNK, 128)
        pltpu.sync_copy(
            w_hbm.at[pl.ds(off, STAGE_CHUNK)], table_s.at[pl.ds(off, STAGE_CHUNK)]
        )

    @pl.when(sid == NUM_SUBCORES - 1)
    def _():
        off = (NUM_SUBCORES - 1) * STAGE_CHUNK
        pltpu.sync_copy(
            w_hbm.at[pl.ds(off, STAGE_TAIL)], table_s.at[pl.ds(off, STAGE_TAIL)]
        )
        pltpu.sync_copy(w_hbm.at[pl.ds(STAGE_REM_OFF, STAGE_REM)], tail_v.at[pl.ds(0, STAGE_REM)])
        pltpu.sync_copy(tail_v.at[pl.ds(0, STAGE_REM)], table_s.at[pl.ds(STAGE_REM_OFF, STAGE_REM)])

    plsc.subcore_barrier()

    gathers = []
    for q in range(NCHUNK):
        in_copies[q].wait()
        cbase = q * CHUNK_COLS
        fbase = q * CHUNK_FLAT

        def compact_row(f, carry, cbase=cbase, fbase=fbase):
            base = fbase + f * CHUNK_COLS
            for v in range(VECS_PER_ROW):
                idx_v[pl.ds(base + v * LANES, LANES)] = xin_v[
                    f, pl.ds(cbase + v * LANES, LANES)
                ]
            return carry

        lax.fori_loop(0, FIELDS, compact_row, 0)
        gathers.append(
            pltpu.async_copy(
                table_s.at[idx_v.at[pl.ds(fbase, CHUNK_FLAT)]],
                vals_v.at[pl.ds(fbase, CHUNK_FLAT)],
                sem_g,
            )
        )

    out_copies = []
    for q in range(NCHUNK):
        gathers[q].wait()
        cbase = q * CHUNK_COLS
        fbase = q * CHUNK_FLAT

        def expand_row(f, carry, cbase=cbase, fbase=fbase):
            base = fbase + f * CHUNK_COLS
            for v in range(VECS_PER_ROW):
                xout_v[f, pl.ds(cbase + v * LANES, LANES)] = vals_v[
                    pl.ds(base + v * LANES, LANES)
                ]
            return carry

        lax.fori_loop(0, FIELDS, expand_row, 0)
        out_copies.append(
            pltpu.async_copy(
                xout_v.at[:, pl.ds(cbase, CHUNK_COLS)],
                out_hbm.at[:, pl.ds(col0 + cbase, CHUNK_COLS)],
                sem_out,
            )
        )

    for c in out_copies:
        c.wait()


def kernel(x, weights):
    out_t = _gather_kernel(weights, x.astype(jnp.int32).T)
    return out_t.T


# speedup vs baseline: 2.1472x; 1.0197x over previous
import functools

import jax
import jax.numpy as jnp
from jax import lax
from jax.experimental import pallas as pl
from jax.experimental.pallas import tpu as pltpu
from jax.experimental.pallas import tpu_sc as plsc

BATCH = 16384
FIELDS = 26
VOCAB = 1000001

NUM_CORES = 2
NUM_SUBCORES = 16
NUM_WORKERS = NUM_CORES * NUM_SUBCORES
COLS_PER_W = BATCH // NUM_WORKERS
B_PER_W = COLS_PER_W * FIELDS

LANES = 16
NCHUNK = 4
CHUNK_COLS = COLS_PER_W // NCHUNK
CHUNK_FLAT = CHUNK_COLS * FIELDS
VECS_PER_ROW = CHUNK_COLS // LANES

STAGE_CHUNK = 62592
STAGE_TAIL = 61056
STAGE_REM_OFF = 15 * STAGE_CHUNK + STAGE_TAIL
STAGE_REM = VOCAB - STAGE_REM_OFF

_mesh = plsc.VectorSubcoreMesh(core_axis_name="c", subcore_axis_name="s")


@functools.partial(
    pl.kernel,
    mesh=_mesh,
    out_type=jax.ShapeDtypeStruct((FIELDS, BATCH), jnp.float32),
    scratch_types=[
        pltpu.VMEM_SHARED((VOCAB,), jnp.float32),
        pltpu.VMEM((FIELDS, COLS_PER_W), jnp.int32),
        pltpu.VMEM((B_PER_W,), jnp.int32),
        pltpu.VMEM((B_PER_W,), jnp.float32),
        pltpu.VMEM((FIELDS, COLS_PER_W), jnp.float32),
        pltpu.VMEM((128,), jnp.float32),
        pltpu.SemaphoreType.DMA,
        pltpu.SemaphoreType.DMA,
        pltpu.SemaphoreType.DMA,
        pltpu.SemaphoreType.DMA,
    ],
    compiler_params=pltpu.CompilerParams(
        use_tc_tiling_on_sc=True, needs_layout_passes=False
    ),
)
def _gather_kernel(
    w_hbm, xt_hbm, out_hbm,
    table_s, xin_v, idx_v, vals_v, xout_v, tail_v,
    sem_in, sem_g, sem_out, sem_t,
):
    cid = lax.axis_index("c")
    sid = lax.axis_index("s")
    wid = sid * NUM_CORES + cid
    col0 = wid * COLS_PER_W

    in_copies = [
        pltpu.async_copy(
            xt_hbm.at[:, pl.ds(col0 + q * CHUNK_COLS, CHUNK_COLS)],
            xin_v.at[:, pl.ds(q * CHUNK_COLS, CHUNK_COLS)],
            sem_in,
        )
        for q in range(NCHUNK)
    ]

    @pl.when(sid < NUM_SUBCORES - 1)
    def _():
        off = pl.multiple_of(sid * STAGE_CHUNK, 128)
        pltpu.async_copy(
            w_hbm.at[pl.ds(off, STAGE_CHUNK)], table_s.at[pl.ds(off, STAGE_CHUNK)],
            sem_t,
        )

    @pl.when(sid == NUM_SUBCORES - 1)
    def _():
        off = (NUM_SUBCORES - 1) * STAGE_CHUNK
        pltpu.async_copy(
            w_hbm.at[pl.ds(off, STAGE_TAIL)], table_s.at[pl.ds(off, STAGE_TAIL)],
            sem_t,
        )
        pltpu.sync_copy(w_hbm.at[pl.ds(STAGE_REM_OFF, STAGE_REM)], tail_v.at[pl.ds(0, STAGE_REM)])
        pltpu.sync_copy(tail_v.at[pl.ds(0, STAGE_REM)], table_s.at[pl.ds(STAGE_REM_OFF, STAGE_REM)])

    for q in range(NCHUNK):
        in_copies[q].wait()
        cbase = q * CHUNK_COLS
        fbase = q * CHUNK_FLAT

        def compact_row(f, carry, cbase=cbase, fbase=fbase):
            base = fbase + f * CHUNK_COLS
            for v in range(VECS_PER_ROW):
                idx_v[pl.ds(base + v * LANES, LANES)] = xin_v[
                    f, pl.ds(cbase + v * LANES, LANES)
                ]
            return carry

        lax.fori_loop(0, FIELDS, compact_row, 0)

    @pl.when(sid < NUM_SUBCORES - 1)
    def _():
        pltpu.make_async_copy(
            w_hbm.at[pl.ds(0, STAGE_CHUNK)], table_s.at[pl.ds(0, STAGE_CHUNK)], sem_t
        ).wait()

    @pl.when(sid == NUM_SUBCORES - 1)
    def _():
        pltpu.make_async_copy(
            w_hbm.at[pl.ds(0, STAGE_TAIL)], table_s.at[pl.ds(0, STAGE_TAIL)], sem_t
        ).wait()

    plsc.subcore_barrier()

    gathers = []
    for q in range(NCHUNK):
        fbase = q * CHUNK_FLAT
        gathers.append(
            pltpu.async_copy(
                table_s.at[idx_v.at[pl.ds(fbase, CHUNK_FLAT)]],
                vals_v.at[pl.ds(fbase, CHUNK_FLAT)],
                sem_g,
            )
        )

    out_copies = []
    for q in range(NCHUNK):
        gathers[q].wait()
        cbase = q * CHUNK_COLS
        fbase = q * CHUNK_FLAT

        def expand_row(f, carry, cbase=cbase, fbase=fbase):
            base = fbase + f * CHUNK_COLS
            for v in range(VECS_PER_ROW):
                xout_v[f, pl.ds(cbase + v * LANES, LANES)] = vals_v[
                    pl.ds(base + v * LANES, LANES)
                ]
            return carry

        lax.fori_loop(0, FIELDS, expand_row, 0)
        out_copies.append(
            pltpu.async_copy(
                xout_v.at[:, pl.ds(cbase, CHUNK_COLS)],
                out_hbm.at[:, pl.ds(col0 + cbase, CHUNK_COLS)],
                sem_out,
            )
        )

    for c in out_copies:
        c.wait()


def kernel(x, weights):
    out_t = _gather_kernel(weights, x.astype(jnp.int32).T)
    return out_t.T
